# baseline (speedup 1.0000x reference)
import functools

import jax
import jax.numpy as jnp
from jax import lax
from jax.experimental import pallas as pl
from jax.experimental.pallas import tpu as pltpu
from jax.experimental.pallas import tpu_sc as plsc

_NC, _NS = 2, 16
_NW = _NC * _NS

_B = 4096 * 200
_D = 64
_CHUNK = 1024
_PER_W = _B // _NW
_N_CHUNKS = _PER_W // _CHUNK

_mesh = plsc.VectorSubcoreMesh(core_axis_name="c", subcore_axis_name="s")


@functools.partial(
    pl.kernel,
    out_type=jax.ShapeDtypeStruct((_B, _D), jnp.float32),
    mesh=_mesh,
    scratch_types=[
        pltpu.VMEM((_CHUNK,), jnp.int32),
        pltpu.VMEM((_CHUNK, _D), jnp.float32),
        pltpu.SemaphoreType.DMA,
    ],
    compiler_params=pltpu.CompilerParams(use_tc_tiling_on_sc=False),
)
def _gather_kernel(tok_hbm, table_hbm, out_hbm, idx_v, rows_v, sem):
    wid = lax.axis_index("s") * _NC + lax.axis_index("c")
    base = wid * _PER_W

    def body(i, carry):
        off = base + i * _CHUNK
        pltpu.sync_copy(tok_hbm.at[pl.ds(off, _CHUNK)], idx_v)
        pltpu.async_copy(table_hbm.at[idx_v], rows_v, sem).wait()
        pltpu.sync_copy(rows_v, out_hbm.at[pl.ds(off, _CHUNK)])
        return carry

    lax.fori_loop(0, _N_CHUNKS, body, 0)


def kernel(tokens, embedding_weight):
    flat = tokens.reshape(_B).astype(jnp.int32)
    out = _gather_kernel(flat, embedding_weight)
    return out.reshape(tokens.shape + (embedding_weight.shape[1],))


# speedup vs baseline: 1.0158x; 1.0158x over previous
import functools

import jax
import jax.numpy as jnp
from jax import lax
from jax.experimental import pallas as pl
from jax.experimental.pallas import tpu as pltpu
from jax.experimental.pallas import tpu_sc as plsc

_NC, _NS = 2, 16
_NW = _NC * _NS

_B = 4096 * 200
_D = 64
_CHUNK = 256
_NBUF = 4
_LAG = 2
_PER_W = _B // _NW
_N_CHUNKS = _PER_W // _CHUNK
_N_OUTER = _N_CHUNKS // _NBUF

_mesh = plsc.VectorSubcoreMesh(core_axis_name="c", subcore_axis_name="s")


@functools.partial(
    pl.kernel,
    out_type=jax.ShapeDtypeStruct((_NW, _N_CHUNKS, _CHUNK, _D), jnp.float32),
    mesh=_mesh,
    scratch_types=[
        pltpu.VMEM((_N_CHUNKS, _CHUNK), jnp.int32),
        pltpu.VMEM((_NBUF, _CHUNK, _D), jnp.float32),
    ] + [pltpu.SemaphoreType.DMA] * (2 * _NBUF),
    compiler_params=pltpu.CompilerParams(use_tc_tiling_on_sc=False),
)
def _gather_kernel(tok_hbm, table_hbm, out_hbm, idx_all, rows, *sems):
    gsem, ssem = sems[:_NBUF], sems[_NBUF:]
    wid = lax.axis_index("s") * _NC + lax.axis_index("c")
    pltpu.sync_copy(tok_hbm.at[wid], idx_all)

    def start_gather(h, b):
        pltpu.async_copy(table_hbm.at[idx_all.at[h]], rows.at[b], gsem[b])

    def wait_gather(h, b):
        pltpu.make_async_copy(table_hbm.at[idx_all.at[h]], rows.at[b],
                              gsem[b]).wait()

    def start_store(h, b):
        pltpu.async_copy(rows.at[b], out_hbm.at[wid, h], ssem[b])

    def wait_store(h, b):
        pltpu.make_async_copy(rows.at[b], out_hbm.at[wid, h], ssem[b]).wait()

    for b in range(_NBUF):
        start_gather(b, b)
        if b >= _LAG:
            wait_gather(b - _LAG, b - _LAG)
            start_store(b - _LAG, b - _LAG)

    def body(p, carry):
        for b in range(_NBUF):
            h = p * _NBUF + b
            bl = (b - _LAG) % _NBUF
            wait_store(h - _NBUF, b)
            start_gather(h, b)
            wait_gather(h - _LAG, bl)
            start_store(h - _LAG, bl)
        return carry

    lax.fori_loop(1, _N_OUTER, body, 0)

    for j in range(_N_CHUNKS - _LAG, _N_CHUNKS):
        b = j % _NBUF
        wait_gather(j, b)
        start_store(j, b)
    for j in range(_N_CHUNKS - _NBUF, _N_CHUNKS):
        wait_store(j, j % _NBUF)


def kernel(tokens, embedding_weight):
    flat = tokens.reshape(_NW, _N_CHUNKS, _CHUNK).astype(jnp.int32)
    out = _gather_kernel(flat, embedding_weight)
    return out.reshape(tokens.shape + (embedding_weight.shape[1],))
